# baseline (speedup 1.0000x reference)
import functools
import jax
import jax.numpy as jnp
from jax import lax
from jax.experimental import pallas as pl
from jax.experimental.pallas import tpu as pltpu
from jax.experimental.pallas import tpu_sc as plsc

NC = 2
NS = 16
NW = NC * NS

N_NODES = 10000
N_EDGES = 320000
EMBED = 128

NODE_PAD = 10240
NODE_PER_W = NODE_PAD // NW
NODE_CHUNKS = ((0, 128), (128, 128), (256, 64))

EDGE_PER_W = N_EDGES // NW
EDGE_CHUNK = 80
EDGE_NCHUNK = EDGE_PER_W // EDGE_CHUNK


def _body(x_hbm, ea_hbm, node_tab, edge_tab, xout, eout,
          nidx_v, nrows_v, eidx_v, erows_v, sem):
    wid = lax.axis_index("s") * NC + lax.axis_index("c")

    pltpu.sync_copy(x_hbm.at[wid], nidx_v)
    nbase = wid * NODE_PER_W
    for off, sz in NODE_CHUNKS:
        pltpu.async_copy(
            node_tab.at[nidx_v.at[pl.ds(off, sz)]],
            nrows_v.at[pl.ds(0, sz)], sem).wait()
        pltpu.sync_copy(nrows_v.at[pl.ds(0, sz)],
                        xout.at[pl.ds(nbase + off, sz)])

    pltpu.sync_copy(ea_hbm.at[wid], eidx_v)
    ebase = wid * EDGE_PER_W

    def chunk(c, carry):
        pltpu.async_copy(edge_tab.at[eidx_v.at[c]], erows_v, sem).wait()
        pltpu.sync_copy(erows_v, eout.at[pl.ds(ebase + c * EDGE_CHUNK,
                                               EDGE_CHUNK)])
        return carry

    lax.fori_loop(0, EDGE_NCHUNK, chunk, 0)


@jax.jit
def kernel(x, edge_index, edge_attr, node_table, edge_table):
    del edge_index
    x_pad = jnp.concatenate(
        [x, jnp.zeros((NODE_PAD - N_NODES,), jnp.int32)]).reshape(NW, NODE_PER_W)
    ea = edge_attr.reshape(NW, EDGE_NCHUNK, EDGE_CHUNK)

    mesh = plsc.VectorSubcoreMesh(core_axis_name="c", subcore_axis_name="s")
    xout, eout = pl.kernel(
        _body,
        out_type=(
            jax.ShapeDtypeStruct((NODE_PAD, EMBED), jnp.float32),
            jax.ShapeDtypeStruct((N_EDGES, EMBED), jnp.float32),
        ),
        mesh=mesh,
        scratch_types=[
            pltpu.VMEM((NODE_PER_W,), jnp.int32),
            pltpu.VMEM((128, EMBED), jnp.float32),
            pltpu.VMEM((EDGE_NCHUNK, EDGE_CHUNK), jnp.int32),
            pltpu.VMEM((EDGE_CHUNK, EMBED), jnp.float32),
            pltpu.SemaphoreType.DMA,
        ],
    )(x_pad, ea, node_table, edge_table)
    return xout[:N_NODES], eout


# speedup vs baseline: 44.0201x; 44.0201x over previous
import functools
import jax
import jax.numpy as jnp
from jax import lax
from jax.experimental import pallas as pl
from jax.experimental.pallas import tpu as pltpu
from jax.experimental.pallas import tpu_sc as plsc

NC = 2
NS = 16
NW = NC * NS

N_NODES = 10000
N_EDGES = 320000
EMBED = 128

NODE_PAD = 10240
NODE_PER_W = NODE_PAD // NW
NODE_CHUNKS = ((0, 128), (128, 128), (256, 64))

EDGE_COLS = N_EDGES // 128
EC = 20
EDGE_GRID = EDGE_COLS // EC


def _node_body(x_hbm, node_tab, xout, nidx_v, nrows_v, sem):
    wid = lax.axis_index("s") * NC + lax.axis_index("c")
    pltpu.sync_copy(x_hbm.at[wid], nidx_v)
    nbase = wid * NODE_PER_W
    for off, sz in NODE_CHUNKS:
        pltpu.async_copy(
            node_tab.at[nidx_v.at[pl.ds(off, sz)]],
            nrows_v.at[pl.ds(0, sz)], sem).wait()
        pltpu.sync_copy(nrows_v.at[pl.ds(0, sz)],
                        xout.at[pl.ds(nbase + off, sz)])


def _edge_body(attr_ref, tab_ref, out_ref):
    row0 = tab_ref[0:1, :]
    row1 = tab_ref[1:2, :]
    for j in range(EC):
        sel = attr_ref[0, :, j:j + 1] == 0
        out_ref[j * 128:(j + 1) * 128, :] = jnp.where(sel, row0, row1)


@jax.jit
def kernel(x, edge_index, edge_attr, node_table, edge_table):
    del edge_index
    x_pad = jnp.concatenate(
        [x, jnp.zeros((NODE_PAD - N_NODES,), jnp.int32)]).reshape(NW, NODE_PER_W)

    mesh = plsc.VectorSubcoreMesh(core_axis_name="c", subcore_axis_name="s")
    xout = pl.kernel(
        _node_body,
        out_type=jax.ShapeDtypeStruct((NODE_PAD, EMBED), jnp.float32),
        mesh=mesh,
        scratch_types=[
            pltpu.VMEM((NODE_PER_W,), jnp.int32),
            pltpu.VMEM((128, EMBED), jnp.float32),
            pltpu.SemaphoreType.DMA,
        ],
    )(x_pad, node_table)

    attr3 = edge_attr.reshape(EDGE_GRID, EC, 128).transpose(0, 2, 1)
    eout = pl.pallas_call(
        _edge_body,
        grid=(EDGE_GRID,),
        in_specs=[
            pl.BlockSpec((1, 128, EC), lambda i: (i, 0, 0)),
            pl.BlockSpec((2, EMBED), lambda i: (0, 0)),
        ],
        out_specs=pl.BlockSpec((EC * 128, EMBED), lambda i: (i, 0)),
        out_shape=jax.ShapeDtypeStruct((N_EDGES, EMBED), jnp.float32),
    )(attr3, edge_table)

    return xout[:N_NODES], eout


# speedup vs baseline: 69.0912x; 1.5695x over previous
import functools
import jax
import jax.numpy as jnp
from jax import lax
from jax.experimental import pallas as pl
from jax.experimental.pallas import tpu as pltpu
from jax.experimental.pallas import tpu_sc as plsc

NC = 2
NS = 16
NW = NC * NS

N_NODES = 10000
N_EDGES = 320000
EMBED = 128

NODE_PER_W = 312
NODE_CHUNKS = ((0, 128), (128, 128), (256, 56))
NODE_TAIL_OFF = NW * NODE_PER_W
NODE_TAIL = N_NODES - NODE_TAIL_OFF

EC = 50
EDGE_GRID = N_EDGES // (EC * 128)


def _node_body(x_hbm, node_tab, xout, nidx_v, nrows_v, tidx_v, sem):
    wid = lax.axis_index("s") * NC + lax.axis_index("c")
    base = wid * NODE_PER_W
    pltpu.sync_copy(x_hbm.at[pl.ds(base, NODE_PER_W)], nidx_v)
    for off, sz in NODE_CHUNKS:
        pltpu.async_copy(
            node_tab.at[nidx_v.at[pl.ds(off, sz)]],
            nrows_v.at[pl.ds(0, sz)], sem).wait()
        pltpu.sync_copy(nrows_v.at[pl.ds(0, sz)],
                        xout.at[pl.ds(base + off, sz)])

    @pl.when(wid == 0)
    def _tail():
        pltpu.sync_copy(x_hbm.at[pl.ds(NODE_TAIL_OFF, NODE_TAIL)], tidx_v)
        pltpu.async_copy(node_tab.at[tidx_v],
                         nrows_v.at[pl.ds(0, NODE_TAIL)], sem).wait()
        pltpu.sync_copy(nrows_v.at[pl.ds(0, NODE_TAIL)],
                        xout.at[pl.ds(NODE_TAIL_OFF, NODE_TAIL)])


def _edge_body(attr_ref, tab_ref, out_ref):
    row0 = tab_ref[0:1, :]
    row1 = tab_ref[1:2, :]
    for j in range(EC):
        sel = attr_ref[0, :, j:j + 1] == 0
        out_ref[j * 128:(j + 1) * 128, :] = jnp.where(sel, row0, row1)


@jax.jit
def kernel(x, edge_index, edge_attr, node_table, edge_table):
    del edge_index

    mesh = plsc.VectorSubcoreMesh(core_axis_name="c", subcore_axis_name="s")
    xout = pl.kernel(
        _node_body,
        out_type=jax.ShapeDtypeStruct((N_NODES, EMBED), jnp.float32),
        mesh=mesh,
        scratch_types=[
            pltpu.VMEM((NODE_PER_W,), jnp.int32),
            pltpu.VMEM((128, EMBED), jnp.float32),
            pltpu.VMEM((NODE_TAIL,), jnp.int32),
            pltpu.SemaphoreType.DMA,
        ],
    )(x, node_table)

    attr3 = edge_attr.reshape(EDGE_GRID, EC, 128).transpose(0, 2, 1)
    eout = pl.pallas_call(
        _edge_body,
        grid=(EDGE_GRID,),
        in_specs=[
            pl.BlockSpec((1, 128, EC), lambda i: (i, 0, 0)),
            pl.BlockSpec((2, EMBED), lambda i: (0, 0)),
        ],
        out_specs=pl.BlockSpec((EC * 128, EMBED), lambda i: (i, 0)),
        out_shape=jax.ShapeDtypeStruct((N_EDGES, EMBED), jnp.float32),
    )(attr3, edge_table)

    return xout, eout


# speedup vs baseline: 78.7787x; 1.1402x over previous
import functools
import jax
import jax.numpy as jnp
from jax import lax
from jax.experimental import pallas as pl
from jax.experimental.pallas import tpu as pltpu
from jax.experimental.pallas import tpu_sc as plsc

NC = 2
NS = 16
NW = NC * NS

N_NODES = 10000
N_EDGES = 320000
EMBED = 128

NODE_PER_W = 312
NODE_CHUNKS = ((0, 128), (128, 128), (256, 56))
NODE_TAIL_OFF = NW * NODE_PER_W
NODE_TAIL = N_NODES - NODE_TAIL_OFF

EC = 100
EDGE_GRID = N_EDGES // (EC * 128)


def _node_body(x_hbm, node_tab, xout, nidx_v, nrows_v, tidx_v, sem):
    wid = lax.axis_index("s") * NC + lax.axis_index("c")
    base = wid * NODE_PER_W
    pltpu.sync_copy(x_hbm.at[pl.ds(base, NODE_PER_W)], nidx_v)
    for off, sz in NODE_CHUNKS:
        pltpu.async_copy(
            node_tab.at[nidx_v.at[pl.ds(off, sz)]],
            nrows_v.at[pl.ds(0, sz)], sem).wait()
        pltpu.sync_copy(nrows_v.at[pl.ds(0, sz)],
                        xout.at[pl.ds(base + off, sz)])

    @pl.when(wid == 0)
    def _tail():
        pltpu.sync_copy(x_hbm.at[pl.ds(NODE_TAIL_OFF, NODE_TAIL)], tidx_v)
        pltpu.async_copy(node_tab.at[tidx_v],
                         nrows_v.at[pl.ds(0, NODE_TAIL)], sem).wait()
        pltpu.sync_copy(nrows_v.at[pl.ds(0, NODE_TAIL)],
                        xout.at[pl.ds(NODE_TAIL_OFF, NODE_TAIL)])


def _edge_body(attr_ref, tab_ref, out_ref):
    row0 = tab_ref[0:1, :]
    row1 = tab_ref[1:2, :]
    for j in range(EC):
        sel = attr_ref[0, :, j:j + 1] == 0
        out_ref[j * 128:(j + 1) * 128, :] = jnp.where(sel, row0, row1)


@jax.jit
def kernel(x, edge_index, edge_attr, node_table, edge_table):
    del edge_index

    mesh = plsc.VectorSubcoreMesh(core_axis_name="c", subcore_axis_name="s")
    xout = pl.kernel(
        _node_body,
        out_type=jax.ShapeDtypeStruct((N_NODES, EMBED), jnp.float32),
        mesh=mesh,
        scratch_types=[
            pltpu.VMEM((NODE_PER_W,), jnp.int32),
            pltpu.VMEM((128, EMBED), jnp.float32),
            pltpu.VMEM((NODE_TAIL,), jnp.int32),
            pltpu.SemaphoreType.DMA,
        ],
    )(x, node_table)

    attr3 = edge_attr.reshape(EDGE_GRID, EC, 128).transpose(0, 2, 1)
    eout = pl.pallas_call(
        _edge_body,
        grid=(EDGE_GRID,),
        in_specs=[
            pl.BlockSpec((1, 128, EC), lambda i: (i, 0, 0)),
            pl.BlockSpec((2, EMBED), lambda i: (0, 0)),
        ],
        out_specs=pl.BlockSpec((EC * 128, EMBED), lambda i: (i, 0)),
        out_shape=jax.ShapeDtypeStruct((N_EDGES, EMBED), jnp.float32),
    )(attr3, edge_table)

    return xout, eout


# speedup vs baseline: 81.0329x; 1.0286x over previous
import functools
import jax
import jax.numpy as jnp
from jax import lax
from jax.experimental import pallas as pl
from jax.experimental.pallas import tpu as pltpu
from jax.experimental.pallas import tpu_sc as plsc

NC = 2
NS = 16
NW = NC * NS

N_NODES = 10000
N_EDGES = 320000
EMBED = 128

NODE_PER_W = 312
NODE_CHUNKS = ((0, 128), (128, 128), (256, 56))
NODE_TAIL_OFF = NW * NODE_PER_W
NODE_TAIL = N_NODES - NODE_TAIL_OFF

EC = 125
EDGE_GRID = N_EDGES // (EC * 128)


def _node_body(x_hbm, node_tab, xout, nidx_v, nrows_v, tidx_v, sem):
    wid = lax.axis_index("s") * NC + lax.axis_index("c")
    base = wid * NODE_PER_W
    pltpu.sync_copy(x_hbm.at[pl.ds(base, NODE_PER_W)], nidx_v)
    for off, sz in NODE_CHUNKS:
        pltpu.async_copy(
            node_tab.at[nidx_v.at[pl.ds(off, sz)]],
            nrows_v.at[pl.ds(0, sz)], sem).wait()
        pltpu.sync_copy(nrows_v.at[pl.ds(0, sz)],
                        xout.at[pl.ds(base + off, sz)])

    @pl.when(wid == 0)
    def _tail():
        pltpu.sync_copy(x_hbm.at[pl.ds(NODE_TAIL_OFF, NODE_TAIL)], tidx_v)
        pltpu.async_copy(node_tab.at[tidx_v],
                         nrows_v.at[pl.ds(0, NODE_TAIL)], sem).wait()
        pltpu.sync_copy(nrows_v.at[pl.ds(0, NODE_TAIL)],
                        xout.at[pl.ds(NODE_TAIL_OFF, NODE_TAIL)])


def _edge_body(attr_ref, tab_ref, out_ref):
    row0 = tab_ref[0:1, :]
    row1 = tab_ref[1:2, :]
    for j in range(EC):
        sel = attr_ref[0, :, j:j + 1] == 0
        out_ref[j * 128:(j + 1) * 128, :] = jnp.where(sel, row0, row1)


@jax.jit
def kernel(x, edge_index, edge_attr, node_table, edge_table):
    del edge_index

    mesh = plsc.VectorSubcoreMesh(core_axis_name="c", subcore_axis_name="s")
    xout = pl.kernel(
        _node_body,
        out_type=jax.ShapeDtypeStruct((N_NODES, EMBED), jnp.float32),
        mesh=mesh,
        scratch_types=[
            pltpu.VMEM((NODE_PER_W,), jnp.int32),
            pltpu.VMEM((128, EMBED), jnp.float32),
            pltpu.VMEM((NODE_TAIL,), jnp.int32),
            pltpu.SemaphoreType.DMA,
        ],
    )(x, node_table)

    attr3 = edge_attr.reshape(EDGE_GRID, EC, 128).transpose(0, 2, 1)
    eout = pl.pallas_call(
        _edge_body,
        grid=(EDGE_GRID,),
        in_specs=[
            pl.BlockSpec((1, 128, EC), lambda i: (i, 0, 0)),
            pl.BlockSpec((2, EMBED), lambda i: (0, 0)),
        ],
        out_specs=pl.BlockSpec((EC * 128, EMBED), lambda i: (i, 0)),
        out_shape=jax.ShapeDtypeStruct((N_EDGES, EMBED), jnp.float32),
    )(attr3, edge_table)

    return xout, eout
